# baseline (speedup 1.0000x reference)
import functools

import jax
import jax.numpy as jnp
from jax.experimental import pallas as pl
from jax.experimental.pallas import tpu as pltpu
from jax.experimental.pallas import tpu_sc as plsc

NUM_EXPERTS = 64
TOP_K = 2
HIDDEN = 1024
INTER = 512
T_TOK = 2048
BM = 64
NBLK = T_TOK * TOP_K // BM + NUM_EXPERTS
MPAD = NBLK * BM


def _router_meta_body(x_ref, rw_ref, pos_ref, wrep_ref, be_ref, nblk_ref):
    x = x_ref[...]
    T = x.shape[0]
    logits = jnp.dot(x, rw_ref[...], preferred_element_type=jnp.float32)
    probs = jax.nn.softmax(logits, axis=-1)
    a1 = jnp.argmax(probs, axis=-1)
    p1 = jnp.max(probs, axis=-1)
    cols = jax.lax.broadcasted_iota(jnp.int32, probs.shape, 1)
    sel1 = cols == a1[:, None]
    masked = jnp.where(sel1, -jnp.inf, probs)
    a2 = jnp.argmax(masked, axis=-1)
    p2 = jnp.max(masked, axis=-1)
    sel2 = cols == a2[:, None]
    s = p1 + p2
    w1 = p1 / s
    w2 = p2 / s

    o1 = sel1.astype(jnp.float32)
    o2 = sel2.astype(jnp.float32)
    ctok = o1 + o2
    CH = 128
    r = jax.lax.broadcasted_iota(jnp.int32, (CH, CH), 0)
    c = jax.lax.broadcasted_iota(jnp.int32, (CH, CH), 1)
    lstrict = (r > c).astype(jnp.float32)
    carry = jnp.zeros((1, NUM_EXPERTS), jnp.float32)
    chunks = []
    for k in range(T // CH):
        chunk = ctok[k * CH:(k + 1) * CH]
        local = jnp.dot(lstrict, chunk, preferred_element_type=jnp.float32)
        chunks.append(local + carry)
        carry = carry + jnp.sum(chunk, axis=0, keepdims=True)
    cprev = jnp.concatenate(chunks, axis=0)

    ones_t = jnp.ones((T, 1), jnp.float32)
    counts_col = jax.lax.dot_general(
        ctok, ones_t, (((0,), (0,)), ((), ())),
        preferred_element_type=jnp.float32)
    nb_col = jnp.floor((counts_col + (BM - 1)) / BM)
    r64 = jax.lax.broadcasted_iota(jnp.int32, (NUM_EXPERTS, NUM_EXPERTS), 0)
    c64 = jax.lax.broadcasted_iota(jnp.int32, (NUM_EXPERTS, NUM_EXPERTS), 1)
    l64 = (r64 > c64).astype(jnp.float32)
    bstart_col = jnp.dot(l64, nb_col, preferred_element_type=jnp.float32)
    nbtot = jnp.sum(nb_col)

    bstart_row = jax.lax.dot_general(
        nb_col, l64, (((0,), (1,)), ((), ())),
        preferred_element_type=jnp.float32)
    posw = cprev + bstart_row * BM
    pos1 = jnp.sum(jnp.where(sel1, posw, 0.0), axis=1)
    pos2 = jnp.sum(jnp.where(sel2, posw, 0.0), axis=1)
    pos_ref[0, :] = pos1.astype(jnp.int32)
    pos_ref[1, :] = pos2.astype(jnp.int32)
    wrep_ref[:T] = jnp.broadcast_to(w1[:, None], (T, 128))
    wrep_ref[T:] = jnp.broadcast_to(w2[:, None], (T, 128))

    ii = jax.lax.broadcasted_iota(jnp.int32, (1, NBLK), 1).astype(jnp.float32)
    ieff = jnp.minimum(ii, nbtot - 1.0)
    mask = (bstart_col <= ieff).astype(jnp.float32)
    be_row = jnp.sum(mask, axis=0) - 1.0
    be_ref[...] = be_row.astype(jnp.int32)
    nblk_ref[0] = nbtot.astype(jnp.int32)


SC_NC = 2
SC_NS = 16
SC_NW = SC_NC * SC_NS
APW = T_TOK * TOP_K // SC_NW
DCH = 64
TPW = T_TOK // SC_NW
CCH = 32

_sc_mesh = plsc.VectorSubcoreMesh(core_axis_name="c", subcore_axis_name="s")


@functools.partial(
    pl.kernel,
    mesh=_sc_mesh,
    out_type=(
        jax.ShapeDtypeStruct((MPAD, HIDDEN), jnp.float32),
        jax.ShapeDtypeStruct((MPAD, 128), jnp.float32),
    ),
    scratch_types=[
        pltpu.VMEM((APW // DCH, DCH), jnp.int32),
        pltpu.VMEM((DCH, HIDDEN), jnp.float32),
        pltpu.VMEM((DCH, 128), jnp.float32),
    ],
)
def _sc_dispatch(x_hbm, pos_hbm, wrep_hbm, xs_hbm, ws_hbm, idx_v, rows_v, wbuf):
    wid = jax.lax.axis_index("s") * SC_NC + jax.lax.axis_index("c")
    tok_base = (wid % (T_TOK // APW)) * APW
    pltpu.sync_copy(pos_hbm.at[wid], idx_v)
    for j in range(APW // DCH):
        pltpu.sync_copy(x_hbm.at[pl.ds(tok_base + j * DCH, DCH)], rows_v)
        pltpu.sync_copy(rows_v, xs_hbm.at[idx_v.at[j]])
        pltpu.sync_copy(wrep_hbm.at[wid, j], wbuf)
        pltpu.sync_copy(wbuf, ws_hbm.at[idx_v.at[j]])


@functools.partial(
    pl.kernel,
    mesh=_sc_mesh,
    out_type=jax.ShapeDtypeStruct((T_TOK, HIDDEN), jnp.float32),
    scratch_types=[
        pltpu.VMEM((TOP_K, TPW // CCH, CCH), jnp.int32),
        pltpu.VMEM((CCH, HIDDEN), jnp.float32),
        pltpu.VMEM((CCH, HIDDEN), jnp.float32),
    ],
)
def _sc_combine(rows_hbm, pos_hbm, out_hbm, idx_v, buf1, buf2):
    wid = jax.lax.axis_index("s") * SC_NC + jax.lax.axis_index("c")
    base = wid * TPW
    pltpu.sync_copy(pos_hbm.at[0, wid], idx_v.at[0])
    pltpu.sync_copy(pos_hbm.at[1, wid], idx_v.at[1])
    for c in range(TPW // CCH):
        pltpu.sync_copy(rows_hbm.at[idx_v.at[0, c]], buf1)
        pltpu.sync_copy(rows_hbm.at[idx_v.at[1, c]], buf2)

        def tok_body(t, _):
            def lane_body(j, _):
                sl = pl.ds(j * 16, 16)
                buf1[t, sl] = buf1[t, sl] + buf2[t, sl]
                return 0

            return jax.lax.fori_loop(0, HIDDEN // 16, lane_body, 0)

        jax.lax.fori_loop(0, CCH, tok_body, 0)
        pltpu.sync_copy(buf1, out_hbm.at[pl.ds(base + c * CCH, CCH)])


def _gmm_body(nblk_ref, be_ref, xs_ref, ws_ref, g_ref, u_ref, d_ref, out_ref):
    i = pl.program_id(0)

    @pl.when(i < nblk_ref[0])
    def _go():
        xb = xs_ref[...]
        g = jnp.dot(xb, g_ref[0], preferred_element_type=jnp.float32)
        u = jnp.dot(xb, u_ref[0], preferred_element_type=jnp.float32)
        a = (g * jax.lax.logistic(g)) * u
        out_ref[...] = (
            jnp.dot(a, d_ref[0], preferred_element_type=jnp.float32)
            * ws_ref[...][:, 0:1]
        )


def _clamp_last(i, nblk_ref, be_ref):
    return (jnp.minimum(i, nblk_ref[0] - 1), 0)


def _wmap(i, nblk_ref, be_ref):
    return (be_ref[jnp.minimum(i, nblk_ref[0] - 1)], 0, 0)


def kernel(hidden_states, router_w, gate_w, up_w, down_w):
    bsz, seq, h = hidden_states.shape
    x = hidden_states.reshape(-1, h)
    T = x.shape[0]

    pos2t, wrep, be, nblk = pl.pallas_call(
        _router_meta_body,
        out_shape=(
            jax.ShapeDtypeStruct((TOP_K, T), jnp.int32),
            jax.ShapeDtypeStruct((TOP_K * T, 128), jnp.float32),
            jax.ShapeDtypeStruct((NBLK,), jnp.int32),
            jax.ShapeDtypeStruct((1,), jnp.int32),
        ),
        out_specs=(
            pl.BlockSpec((TOP_K, T), lambda: (0, 0)),
            pl.BlockSpec((TOP_K * T, 128), lambda: (0, 0)),
            pl.BlockSpec((NBLK,), lambda: (0,)),
            pl.BlockSpec(memory_space=pltpu.SMEM),
        ),
    )(x, router_w)

    pos_w = pos2t.reshape(SC_NW, APW // DCH, DCH)
    wrep_w = wrep.reshape(SC_NW, APW // DCH, DCH, 128)
    xs, ws = _sc_dispatch(x, pos_w, wrep_w)

    rows = pl.pallas_call(
        _gmm_body,
        grid_spec=pltpu.PrefetchScalarGridSpec(
            num_scalar_prefetch=2,
            grid=(NBLK,),
            in_specs=[
                pl.BlockSpec((BM, HIDDEN), _clamp_last),
                pl.BlockSpec((BM, 128), _clamp_last),
                pl.BlockSpec((1, HIDDEN, INTER), _wmap),
                pl.BlockSpec((1, HIDDEN, INTER), _wmap),
                pl.BlockSpec((1, INTER, HIDDEN), _wmap),
            ],
            out_specs=pl.BlockSpec((BM, HIDDEN), _clamp_last),
        ),
        out_shape=jax.ShapeDtypeStruct((MPAD, HIDDEN), jnp.float32),
    )(nblk, be, xs, ws,
      gate_w.astype(jnp.bfloat16), up_w.astype(jnp.bfloat16),
      down_w.astype(jnp.bfloat16))

    pos_c = pos2t.reshape(TOP_K, SC_NW, TPW // CCH, CCH)
    final = _sc_combine(rows, pos_c)

    router_logits = jnp.zeros((bsz * seq, NUM_EXPERTS), dtype=hidden_states.dtype)
    return (final.reshape(bsz, seq, h), router_logits)


# speedup vs baseline: 1.4798x; 1.4798x over previous
import functools

import jax
import jax.numpy as jnp
from jax.experimental import pallas as pl
from jax.experimental.pallas import tpu as pltpu
from jax.experimental.pallas import tpu_sc as plsc

NUM_EXPERTS = 64
TOP_K = 2
HIDDEN = 1024
INTER = 512
T_TOK = 2048
BM = 64
NBLK = T_TOK * TOP_K // BM + NUM_EXPERTS
MPAD = NBLK * BM


def _router_meta_body(x_ref, rw_ref, pos_ref, wrep_ref, be_ref, nblk_ref):
    x = x_ref[...]
    T = x.shape[0]
    logits = jnp.dot(x, rw_ref[...], preferred_element_type=jnp.float32)
    probs = jax.nn.softmax(logits, axis=-1)
    a1 = jnp.argmax(probs, axis=-1)
    p1 = jnp.max(probs, axis=-1)
    cols = jax.lax.broadcasted_iota(jnp.int32, probs.shape, 1)
    sel1 = cols == a1[:, None]
    masked = jnp.where(sel1, -jnp.inf, probs)
    a2 = jnp.argmax(masked, axis=-1)
    p2 = jnp.max(masked, axis=-1)
    sel2 = cols == a2[:, None]
    s = p1 + p2
    w1 = p1 / s
    w2 = p2 / s

    o1 = sel1.astype(jnp.float32)
    o2 = sel2.astype(jnp.float32)
    ctok = o1 + o2
    CH = 128
    r = jax.lax.broadcasted_iota(jnp.int32, (CH, CH), 0)
    c = jax.lax.broadcasted_iota(jnp.int32, (CH, CH), 1)
    lstrict = (r > c).astype(jnp.float32)
    carry = jnp.zeros((1, NUM_EXPERTS), jnp.float32)
    chunks = []
    for k in range(T // CH):
        chunk = ctok[k * CH:(k + 1) * CH]
        local = jnp.dot(lstrict, chunk, preferred_element_type=jnp.float32)
        chunks.append(local + carry)
        carry = carry + jnp.sum(chunk, axis=0, keepdims=True)
    cprev = jnp.concatenate(chunks, axis=0)

    ones_t = jnp.ones((T, 1), jnp.float32)
    counts_col = jax.lax.dot_general(
        ctok, ones_t, (((0,), (0,)), ((), ())),
        preferred_element_type=jnp.float32)
    nb_col = jnp.floor((counts_col + (BM - 1)) / BM)
    r64 = jax.lax.broadcasted_iota(jnp.int32, (NUM_EXPERTS, NUM_EXPERTS), 0)
    c64 = jax.lax.broadcasted_iota(jnp.int32, (NUM_EXPERTS, NUM_EXPERTS), 1)
    l64 = (r64 > c64).astype(jnp.float32)
    bstart_col = jnp.dot(l64, nb_col, preferred_element_type=jnp.float32)
    nbtot = jnp.sum(nb_col)

    bstart_row = jax.lax.dot_general(
        nb_col, l64, (((0,), (1,)), ((), ())),
        preferred_element_type=jnp.float32)
    posw = cprev + bstart_row * BM
    pos1 = jnp.sum(jnp.where(sel1, posw, 0.0), axis=1)
    pos2 = jnp.sum(jnp.where(sel2, posw, 0.0), axis=1)
    pos_ref[0, :] = pos1.astype(jnp.int32)
    pos_ref[1, :] = pos2.astype(jnp.int32)
    wrep_ref[:T] = jnp.broadcast_to(w1[:, None], (T, 128))
    wrep_ref[T:] = jnp.broadcast_to(w2[:, None], (T, 128))

    ii = jax.lax.broadcasted_iota(jnp.int32, (1, NBLK), 1).astype(jnp.float32)
    ieff = jnp.minimum(ii, nbtot - 1.0)
    mask = (bstart_col <= ieff).astype(jnp.float32)
    be_row = jnp.sum(mask, axis=0) - 1.0
    be_ref[...] = be_row.astype(jnp.int32)
    nblk_ref[0] = nbtot.astype(jnp.int32)


SC_NC = 2
SC_NS = 16
SC_NW = SC_NC * SC_NS
APW = T_TOK * TOP_K // SC_NW
DCH = 32
DNC = APW // DCH
TPW = T_TOK // SC_NW
CCH = 16
CNC = TPW // CCH

_sc_mesh = plsc.VectorSubcoreMesh(core_axis_name="c", subcore_axis_name="s")


@functools.partial(
    pl.kernel,
    mesh=_sc_mesh,
    out_type=(
        jax.ShapeDtypeStruct((MPAD, HIDDEN), jnp.float32),
        jax.ShapeDtypeStruct((MPAD, 128), jnp.float32),
    ),
    scratch_types=[
        pltpu.VMEM((DNC, DCH), jnp.int32),
        pltpu.VMEM((2, DCH, HIDDEN), jnp.float32),
        pltpu.VMEM((2, DCH, 128), jnp.float32),
        pltpu.SemaphoreType.DMA((2,)),
        pltpu.SemaphoreType.DMA((2,)),
        pltpu.SemaphoreType.DMA((2,)),
        pltpu.SemaphoreType.DMA((2,)),
    ],
)
def _sc_dispatch(x_hbm, pos_hbm, wrep_hbm, xs_hbm, ws_hbm,
                 idx_v, rows_v, wbuf, ldr_s, ldw_s, scr_s, scw_s):
    wid = jax.lax.axis_index("s") * SC_NC + jax.lax.axis_index("c")
    tok_base = (wid % (T_TOK // APW)) * APW
    pltpu.sync_copy(pos_hbm.at[wid], idx_v)

    def start_load(j, b):
        return (
            pltpu.async_copy(
                x_hbm.at[pl.ds(tok_base + j * DCH, DCH)], rows_v.at[b],
                ldr_s.at[b]),
            pltpu.async_copy(wrep_hbm.at[wid, j], wbuf.at[b], ldw_s.at[b]),
        )

    loads = {0: start_load(0, 0)}
    scats = {}
    for j in range(DNC):
        b = j % 2
        if j + 1 < DNC:
            if j - 1 >= 0:
                for h in scats[j - 1]:
                    h.wait()
            loads[j + 1] = start_load(j + 1, b ^ 1)
        for h in loads[j]:
            h.wait()
        scats[j] = (
            pltpu.async_copy(rows_v.at[b], xs_hbm.at[idx_v.at[j]], scr_s.at[b]),
            pltpu.async_copy(wbuf.at[b], ws_hbm.at[idx_v.at[j]], scw_s.at[b]),
        )
    for j in (DNC - 2, DNC - 1):
        for h in scats[j]:
            h.wait()


@functools.partial(
    pl.kernel,
    mesh=_sc_mesh,
    out_type=jax.ShapeDtypeStruct((T_TOK, HIDDEN), jnp.float32),
    scratch_types=[
        pltpu.VMEM((TOP_K, CNC, CCH), jnp.int32),
        pltpu.VMEM((2, CCH, HIDDEN), jnp.float32),
        pltpu.VMEM((2, CCH, HIDDEN), jnp.float32),
        pltpu.SemaphoreType.DMA((2,)),
        pltpu.SemaphoreType.DMA((2,)),
    ],
)
def _sc_combine(rows_hbm, pos_hbm, out_hbm, idx_v, buf1, buf2, g1_s, g2_s):
    wid = jax.lax.axis_index("s") * SC_NC + jax.lax.axis_index("c")
    base = wid * TPW
    pltpu.sync_copy(pos_hbm.at[0, wid], idx_v.at[0])
    pltpu.sync_copy(pos_hbm.at[1, wid], idx_v.at[1])

    def start_gather(c, b):
        return (
            pltpu.async_copy(rows_hbm.at[idx_v.at[0, c]], buf1.at[b],
                             g1_s.at[b]),
            pltpu.async_copy(rows_hbm.at[idx_v.at[1, c]], buf2.at[b],
                             g2_s.at[b]),
        )

    gath = {0: start_gather(0, 0)}
    for c in range(CNC):
        b = c % 2
        if c + 1 < CNC:
            gath[c + 1] = start_gather(c + 1, b ^ 1)
        for h in gath[c]:
            h.wait()
        for t in range(CCH):
            def lane_body(j, _, _t=t, _b=b):
                sl = pl.ds(j * 16, 16)
                buf1[_b, _t, sl] = buf1[_b, _t, sl] + buf2[_b, _t, sl]
                return 0

            jax.lax.fori_loop(0, HIDDEN // 16, lane_body, 0, unroll=8)
        pltpu.sync_copy(buf1.at[b], out_hbm.at[pl.ds(base + c * CCH, CCH)])


def _gmm_body(nblk_ref, be_ref, xs_ref, ws_ref, g_ref, u_ref, d_ref, out_ref):
    i = pl.program_id(0)

    @pl.when(i < nblk_ref[0])
    def _go():
        xb = xs_ref[...]
        g = jnp.dot(xb, g_ref[0], preferred_element_type=jnp.float32)
        u = jnp.dot(xb, u_ref[0], preferred_element_type=jnp.float32)
        a = (g * jax.lax.logistic(g)) * u
        out_ref[...] = (
            jnp.dot(a, d_ref[0], preferred_element_type=jnp.float32)
            * ws_ref[...][:, 0:1]
        )


def _clamp_last(i, nblk_ref, be_ref):
    return (jnp.minimum(i, nblk_ref[0] - 1), 0)


def _wmap(i, nblk_ref, be_ref):
    return (be_ref[jnp.minimum(i, nblk_ref[0] - 1)], 0, 0)


def kernel(hidden_states, router_w, gate_w, up_w, down_w):
    bsz, seq, h = hidden_states.shape
    x = hidden_states.reshape(-1, h)
    T = x.shape[0]

    pos2t, wrep, be, nblk = pl.pallas_call(
        _router_meta_body,
        out_shape=(
            jax.ShapeDtypeStruct((TOP_K, T), jnp.int32),
            jax.ShapeDtypeStruct((TOP_K * T, 128), jnp.float32),
            jax.ShapeDtypeStruct((NBLK,), jnp.int32),
            jax.ShapeDtypeStruct((1,), jnp.int32),
        ),
        out_specs=(
            pl.BlockSpec((TOP_K, T), lambda: (0, 0)),
            pl.BlockSpec((TOP_K * T, 128), lambda: (0, 0)),
            pl.BlockSpec((NBLK,), lambda: (0,)),
            pl.BlockSpec(memory_space=pltpu.SMEM),
        ),
    )(x, router_w)

    pos_w = pos2t.reshape(SC_NW, APW // DCH, DCH)
    wrep_w = wrep.reshape(SC_NW, APW // DCH, DCH, 128)
    xs, ws = _sc_dispatch(x, pos_w, wrep_w)

    rows = pl.pallas_call(
        _gmm_body,
        grid_spec=pltpu.PrefetchScalarGridSpec(
            num_scalar_prefetch=2,
            grid=(NBLK,),
            in_specs=[
                pl.BlockSpec((BM, HIDDEN), _clamp_last),
                pl.BlockSpec((BM, 128), _clamp_last),
                pl.BlockSpec((1, HIDDEN, INTER), _wmap),
                pl.BlockSpec((1, HIDDEN, INTER), _wmap),
                pl.BlockSpec((1, INTER, HIDDEN), _wmap),
            ],
            out_specs=pl.BlockSpec((BM, HIDDEN), _clamp_last),
        ),
        out_shape=jax.ShapeDtypeStruct((MPAD, HIDDEN), jnp.float32),
    )(nblk, be, xs, ws, gate_w, up_w, down_w)

    pos_c = pos2t.reshape(TOP_K, SC_NW, TPW // CCH, CCH)
    final = _sc_combine(rows, pos_c)

    router_logits = jnp.zeros((bsz * seq, NUM_EXPERTS), dtype=hidden_states.dtype)
    return (final.reshape(bsz, seq, h), router_logits)


# speedup vs baseline: 1.5023x; 1.0152x over previous
import functools

import jax
import jax.numpy as jnp
from jax.experimental import pallas as pl
from jax.experimental.pallas import tpu as pltpu
from jax.experimental.pallas import tpu_sc as plsc

NUM_EXPERTS = 64
TOP_K = 2
HIDDEN = 1024
INTER = 512
T_TOK = 2048
BM = 64
NBLK = T_TOK * TOP_K // BM + NUM_EXPERTS
MPAD = NBLK * BM


def _router_meta_body(x_ref, rw_ref, pos_ref, wrep_ref, be_ref, nblk_ref):
    x = x_ref[...]
    T = x.shape[0]
    logits = jnp.dot(x, rw_ref[...], preferred_element_type=jnp.float32)
    probs = jax.nn.softmax(logits, axis=-1)
    a1 = jnp.argmax(probs, axis=-1)
    p1 = jnp.max(probs, axis=-1)
    cols = jax.lax.broadcasted_iota(jnp.int32, probs.shape, 1)
    sel1 = cols == a1[:, None]
    masked = jnp.where(sel1, -jnp.inf, probs)
    a2 = jnp.argmax(masked, axis=-1)
    p2 = jnp.max(masked, axis=-1)
    sel2 = cols == a2[:, None]
    s = p1 + p2
    w1 = p1 / s
    w2 = p2 / s

    o1 = sel1.astype(jnp.float32)
    o2 = sel2.astype(jnp.float32)
    ctok = o1 + o2
    CH = 128
    r = jax.lax.broadcasted_iota(jnp.int32, (CH, CH), 0)
    c = jax.lax.broadcasted_iota(jnp.int32, (CH, CH), 1)
    lstrict = (r > c).astype(jnp.float32)
    carry = jnp.zeros((1, NUM_EXPERTS), jnp.float32)
    chunks = []
    for k in range(T // CH):
        chunk = ctok[k * CH:(k + 1) * CH]
        local = jnp.dot(lstrict, chunk, preferred_element_type=jnp.float32)
        chunks.append(local + carry)
        carry = carry + jnp.sum(chunk, axis=0, keepdims=True)
    cprev = jnp.concatenate(chunks, axis=0)

    ones_t = jnp.ones((T, 1), jnp.float32)
    counts_col = jax.lax.dot_general(
        ctok, ones_t, (((0,), (0,)), ((), ())),
        preferred_element_type=jnp.float32)
    nb_col = jnp.floor((counts_col + (BM - 1)) / BM)
    r64 = jax.lax.broadcasted_iota(jnp.int32, (NUM_EXPERTS, NUM_EXPERTS), 0)
    c64 = jax.lax.broadcasted_iota(jnp.int32, (NUM_EXPERTS, NUM_EXPERTS), 1)
    l64 = (r64 > c64).astype(jnp.float32)
    bstart_col = jnp.dot(l64, nb_col, preferred_element_type=jnp.float32)
    nbtot = jnp.sum(nb_col)

    bstart_row = jax.lax.dot_general(
        nb_col, l64, (((0,), (1,)), ((), ())),
        preferred_element_type=jnp.float32)
    posw = cprev + bstart_row * BM
    pos1 = jnp.sum(jnp.where(sel1, posw, 0.0), axis=1)
    pos2 = jnp.sum(jnp.where(sel2, posw, 0.0), axis=1)
    pos_ref[0, :] = pos1.astype(jnp.int32)
    pos_ref[1, :] = pos2.astype(jnp.int32)
    wrep_ref[:T] = jnp.broadcast_to(w1[:, None], (T, 128))
    wrep_ref[T:] = jnp.broadcast_to(w2[:, None], (T, 128))

    ii = jax.lax.broadcasted_iota(jnp.int32, (1, NBLK), 1).astype(jnp.float32)
    ieff = jnp.minimum(ii, nbtot - 1.0)
    mask = (bstart_col <= ieff).astype(jnp.float32)
    be_row = jnp.sum(mask, axis=0) - 1.0
    be_ref[...] = be_row.astype(jnp.int32)
    nblk_ref[0] = nbtot.astype(jnp.int32)


SC_NC = 2
SC_NS = 16
SC_NW = SC_NC * SC_NS
TPW = T_TOK // SC_NW
DCH = 32
DNC = TPW // DCH
CCH = 16
CNC = TPW // CCH

_sc_mesh = plsc.VectorSubcoreMesh(core_axis_name="c", subcore_axis_name="s")


@functools.partial(
    pl.kernel,
    mesh=_sc_mesh,
    out_type=(
        jax.ShapeDtypeStruct((MPAD, HIDDEN), jnp.float32),
        jax.ShapeDtypeStruct((MPAD, 128), jnp.float32),
    ),
    scratch_types=[
        pltpu.VMEM((TOP_K, DNC, DCH), jnp.int32),
        pltpu.VMEM((2, DCH, HIDDEN), jnp.float32),
        pltpu.VMEM((2, TOP_K, DCH, 128), jnp.float32),
        pltpu.SemaphoreType.DMA((2,)),
        pltpu.SemaphoreType.DMA((2,)),
        pltpu.SemaphoreType.DMA((2,)),
        pltpu.SemaphoreType.DMA((2,)),
    ],
)
def _sc_dispatch(x_hbm, pos_hbm, wrep_hbm, xs_hbm, ws_hbm,
                 idx_v, rows_v, wbuf, ldr_s, ldw_s, scr_s, scw_s):
    wid = jax.lax.axis_index("s") * SC_NC + jax.lax.axis_index("c")
    tok_base = wid * TPW
    pltpu.sync_copy(pos_hbm.at[0, wid], idx_v.at[0])
    pltpu.sync_copy(pos_hbm.at[1, wid], idx_v.at[1])

    def start_load(j, b):
        return (
            pltpu.async_copy(
                x_hbm.at[pl.ds(tok_base + j * DCH, DCH)], rows_v.at[b],
                ldr_s.at[b]),
            pltpu.async_copy(wrep_hbm.at[0, wid, j], wbuf.at[b, 0],
                             ldw_s.at[b]),
            pltpu.async_copy(wrep_hbm.at[1, wid, j], wbuf.at[b, 1],
                             ldw_s.at[b]),
        )

    loads = {0: start_load(0, 0)}
    scats = {}
    for j in range(DNC):
        b = j % 2
        if j + 1 < DNC:
            if j - 1 >= 0:
                for h in scats[j - 1]:
                    h.wait()
            loads[j + 1] = start_load(j + 1, b ^ 1)
        for h in loads[j]:
            h.wait()
        scats[j] = (
            pltpu.async_copy(rows_v.at[b], xs_hbm.at[idx_v.at[0, j]],
                             scr_s.at[b]),
            pltpu.async_copy(rows_v.at[b], xs_hbm.at[idx_v.at[1, j]],
                             scr_s.at[b]),
            pltpu.async_copy(wbuf.at[b, 0], ws_hbm.at[idx_v.at[0, j]],
                             scw_s.at[b]),
            pltpu.async_copy(wbuf.at[b, 1], ws_hbm.at[idx_v.at[1, j]],
                             scw_s.at[b]),
        )
    for j in (DNC - 2, DNC - 1):
        for h in scats[j]:
            h.wait()


@functools.partial(
    pl.kernel,
    mesh=_sc_mesh,
    out_type=jax.ShapeDtypeStruct((T_TOK, HIDDEN), jnp.float32),
    scratch_types=[
        pltpu.VMEM((TOP_K, CNC, CCH), jnp.int32),
        pltpu.VMEM((2, CCH, HIDDEN), jnp.float32),
        pltpu.VMEM((2, CCH, HIDDEN), jnp.float32),
        pltpu.SemaphoreType.DMA((2,)),
        pltpu.SemaphoreType.DMA((2,)),
    ],
)
def _sc_combine(rows_hbm, pos_hbm, out_hbm, idx_v, buf1, buf2, g1_s, g2_s):
    wid = jax.lax.axis_index("s") * SC_NC + jax.lax.axis_index("c")
    base = wid * TPW
    pltpu.sync_copy(pos_hbm.at[0, wid], idx_v.at[0])
    pltpu.sync_copy(pos_hbm.at[1, wid], idx_v.at[1])

    def start_gather(c, b):
        return (
            pltpu.async_copy(rows_hbm.at[idx_v.at[0, c]], buf1.at[b],
                             g1_s.at[b]),
            pltpu.async_copy(rows_hbm.at[idx_v.at[1, c]], buf2.at[b],
                             g2_s.at[b]),
        )

    gath = {0: start_gather(0, 0)}
    for c in range(CNC):
        b = c % 2
        if c + 1 < CNC:
            gath[c + 1] = start_gather(c + 1, b ^ 1)
        for h in gath[c]:
            h.wait()
        for t in range(CCH):
            def lane_body(j, _, _t=t, _b=b):
                sl = pl.ds(j * 16, 16)
                buf1[_b, _t, sl] = buf1[_b, _t, sl] + buf2[_b, _t, sl]
                return 0

            jax.lax.fori_loop(0, HIDDEN // 16, lane_body, 0, unroll=8)
        pltpu.sync_copy(buf1.at[b], out_hbm.at[pl.ds(base + c * CCH, CCH)])


def _gmm_body(nblk_ref, be_ref, xs_ref, ws_ref, g_ref, u_ref, d_ref, out_ref):
    i = pl.program_id(0)

    @pl.when(i < nblk_ref[0])
    def _go():
        xb = xs_ref[...]
        g = jnp.dot(xb, g_ref[0], preferred_element_type=jnp.float32)
        u = jnp.dot(xb, u_ref[0], preferred_element_type=jnp.float32)
        a = (g * jax.lax.logistic(g)) * u
        out_ref[...] = (
            jnp.dot(a, d_ref[0], preferred_element_type=jnp.float32)
            * ws_ref[...][:, 0:1]
        )


def _clamp_last(i, nblk_ref, be_ref):
    return (jnp.minimum(i, nblk_ref[0] - 1), 0)


def _wmap(i, nblk_ref, be_ref):
    return (be_ref[jnp.minimum(i, nblk_ref[0] - 1)], 0, 0)


def kernel(hidden_states, router_w, gate_w, up_w, down_w):
    bsz, seq, h = hidden_states.shape
    x = hidden_states.reshape(-1, h)
    T = x.shape[0]

    pos2t, wrep, be, nblk = pl.pallas_call(
        _router_meta_body,
        out_shape=(
            jax.ShapeDtypeStruct((TOP_K, T), jnp.int32),
            jax.ShapeDtypeStruct((TOP_K * T, 128), jnp.float32),
            jax.ShapeDtypeStruct((NBLK,), jnp.int32),
            jax.ShapeDtypeStruct((1,), jnp.int32),
        ),
        out_specs=(
            pl.BlockSpec((TOP_K, T), lambda: (0, 0)),
            pl.BlockSpec((TOP_K * T, 128), lambda: (0, 0)),
            pl.BlockSpec((NBLK,), lambda: (0,)),
            pl.BlockSpec(memory_space=pltpu.SMEM),
        ),
    )(x, router_w)

    pos_w = pos2t.reshape(TOP_K, SC_NW, DNC, DCH)
    wrep_w = wrep.reshape(TOP_K, SC_NW, DNC, DCH, 128)
    xs, ws = _sc_dispatch(x, pos_w, wrep_w)

    rows = pl.pallas_call(
        _gmm_body,
        grid_spec=pltpu.PrefetchScalarGridSpec(
            num_scalar_prefetch=2,
            grid=(NBLK,),
            in_specs=[
                pl.BlockSpec((BM, HIDDEN), _clamp_last),
                pl.BlockSpec((BM, 128), _clamp_last),
                pl.BlockSpec((1, HIDDEN, INTER), _wmap),
                pl.BlockSpec((1, HIDDEN, INTER), _wmap),
                pl.BlockSpec((1, INTER, HIDDEN), _wmap),
            ],
            out_specs=pl.BlockSpec((BM, HIDDEN), _clamp_last),
        ),
        out_shape=jax.ShapeDtypeStruct((MPAD, HIDDEN), jnp.float32),
    )(nblk, be, xs, ws, gate_w, up_w, down_w)

    pos_c = pos2t.reshape(TOP_K, SC_NW, TPW // CCH, CCH)
    final = _sc_combine(rows, pos_c)

    router_logits = jnp.zeros((bsz * seq, NUM_EXPERTS), dtype=hidden_states.dtype)
    return (final.reshape(bsz, seq, h), router_logits)
